# baseline (speedup 1.0000x reference)
from functools import partial

import jax
import jax.numpy as jnp
from jax import lax
from jax.experimental import pallas as pl
from jax.experimental.pallas import tpu as pltpu

H = 128
SUB = 8
LPAD = 32
NC = 128


def _lstm_kernel(ids_ref, lens_ref, table_ref, wih_ref, whh_ref, b_ref,
                 out_ref, xg_ref, *, bg, chains):
    B = bg * SUB
    BGC = bg // chains
    BC = BGC * SUB

    tw = (jnp.dot(table_ref[...], wih_ref[...],
                  preferred_element_type=jnp.float32)
          + b_ref[...]).astype(jnp.bfloat16)

    iota = lax.broadcasted_iota(jnp.int32, (1, NC), 1)
    ids = ids_ref[...].reshape(bg * LPAD * SUB, 1)
    onehot = jnp.where(ids == iota, 1.0, 0.0).astype(jnp.bfloat16)
    xg = jnp.dot(onehot, tw, preferred_element_type=jnp.float32)
    xg_ref[...] = xg.astype(jnp.bfloat16).reshape(bg, LPAD * SUB, 4 * H)

    lens = lens_ref[...].reshape(B, 1)

    def step(t, carry):
        out = []
        for k in range(chains):
            h, c, hout = carry[k]
            xs = xg_ref[k * BGC:(k + 1) * BGC, pl.ds(t * SUB, SUB), :]
            gates = xs.reshape(BC, 4 * H).astype(jnp.float32) + jnp.dot(
                h.astype(jnp.bfloat16), whh_ref[...],
                preferred_element_type=jnp.float32)
            sig = 1.0 / (1.0 + jnp.exp(-gates[:, :3 * H]))
            g = jnp.tanh(gates[:, 3 * H:])
            i, f, o = sig[:, :H], sig[:, H:2 * H], sig[:, 2 * H:]
            c_new = f * c + i * g
            h_new = o * jnp.tanh(c_new)
            last = t == lens[k * BC:(k + 1) * BC] - 1
            out.append((h_new, c_new, jnp.where(last, h_new, hout)))
        return tuple(out)

    zeros = lambda: jnp.zeros((BC, H), jnp.float32)
    init = tuple((zeros(), zeros(), zeros()) for _ in range(chains))
    final = lax.fori_loop(0, LPAD, step, init, unroll=True)
    for k in range(chains):
        out_ref[k * BGC:(k + 1) * BGC, :, :] = final[k][2].reshape(BGC, SUB, H)


@partial(jax.jit, static_argnames=("bg",))
def _run(ids, lens, table, wih, whh, b, *, bg):
    G = ids.shape[0]
    GB = G // bg
    rows = LPAD * SUB
    return pl.pallas_call(
        partial(_lstm_kernel, bg=bg, chains=2),
        grid=(GB,),
        in_specs=[
            pl.BlockSpec((bg, rows, 1), lambda g: (g, 0, 0)),
            pl.BlockSpec((bg, SUB, 1), lambda g: (g, 0, 0)),
            pl.BlockSpec((NC, H), lambda g: (0, 0)),
            pl.BlockSpec((H, 4 * H), lambda g: (0, 0)),
            pl.BlockSpec((H, 4 * H), lambda g: (0, 0)),
            pl.BlockSpec((1, 4 * H), lambda g: (0, 0)),
        ],
        out_specs=pl.BlockSpec((bg, SUB, H), lambda g: (g, 0, 0)),
        out_shape=jax.ShapeDtypeStruct((G, SUB, H), jnp.float32),
        scratch_shapes=[pltpu.VMEM((bg, rows, 4 * H), jnp.bfloat16)],
        compiler_params=pltpu.CompilerParams(
            dimension_semantics=("parallel",)),
    )(ids, lens, table, wih, whh, b)


def kernel(maxlen, ids, lens, table, wih, whh, b):
    G = ids.shape[0]
    bg = 32
    while G % bg:
        bg //= 2
    return _run(ids, lens, table, wih, whh, b, bg=bg)


# speedup vs baseline: 1.0882x; 1.0882x over previous
from functools import partial

import jax
import jax.numpy as jnp
from jax import lax
from jax.experimental import pallas as pl
from jax.experimental.pallas import tpu as pltpu

H = 128
SUB = 8
LPAD = 32
NC = 128


def _lstm_kernel(ids_ref, lens_ref, table_ref, wih_ref, whh_ref, b_ref,
                 out_ref, xg_ref, *, bg, chains):
    B = bg * SUB
    BGC = bg // chains
    BC = BGC * SUB

    tw = (jnp.dot(table_ref[...], wih_ref[...],
                  preferred_element_type=jnp.float32)
          + b_ref[...]).astype(jnp.bfloat16)

    iota = lax.broadcasted_iota(jnp.int32, (1, NC), 1)
    ids = ids_ref[...].reshape(bg * LPAD * SUB, 1)
    onehot = jnp.where(ids == iota, 1.0, 0.0).astype(jnp.bfloat16)
    xg = jnp.dot(onehot, tw, preferred_element_type=jnp.float32)
    xg_ref[...] = xg.reshape(bg, LPAD * SUB, 4 * H)

    lens = lens_ref[...].reshape(B, 1)

    def step(t, carry):
        out = []
        for k in range(chains):
            h, c, hout = carry[k]
            xs = xg_ref[k * BGC:(k + 1) * BGC, pl.ds(t * SUB, SUB), :]
            gates = xs.reshape(BC, 4 * H) + jnp.dot(
                h.astype(jnp.bfloat16), whh_ref[...],
                preferred_element_type=jnp.float32)
            sig = jax.nn.sigmoid(gates[:, :3 * H])
            g = jnp.tanh(gates[:, 3 * H:])
            i, f, o = sig[:, :H], sig[:, H:2 * H], sig[:, 2 * H:]
            c_new = f * c + i * g
            h_new = o * jnp.tanh(c_new)
            last = t == lens[k * BC:(k + 1) * BC] - 1
            out.append((h_new, c_new, jnp.where(last, h_new, hout)))
        return tuple(out)

    zeros = lambda: jnp.zeros((BC, H), jnp.float32)
    init = tuple((zeros(), zeros(), zeros()) for _ in range(chains))
    final = lax.fori_loop(0, LPAD, step, init, unroll=True)
    for k in range(chains):
        out_ref[k * BGC:(k + 1) * BGC, :, :] = final[k][2].reshape(BGC, SUB, H)


@partial(jax.jit, static_argnames=("bg",))
def _run(ids, lens, table, wih, whh, b, *, bg):
    G = ids.shape[0]
    GB = G // bg
    rows = LPAD * SUB
    return pl.pallas_call(
        partial(_lstm_kernel, bg=bg, chains=2),
        grid=(GB,),
        in_specs=[
            pl.BlockSpec((bg, rows, 1), lambda g: (g, 0, 0)),
            pl.BlockSpec((bg, SUB, 1), lambda g: (g, 0, 0)),
            pl.BlockSpec((NC, H), lambda g: (0, 0)),
            pl.BlockSpec((H, 4 * H), lambda g: (0, 0)),
            pl.BlockSpec((H, 4 * H), lambda g: (0, 0)),
            pl.BlockSpec((1, 4 * H), lambda g: (0, 0)),
        ],
        out_specs=pl.BlockSpec((bg, SUB, H), lambda g: (g, 0, 0)),
        out_shape=jax.ShapeDtypeStruct((G, SUB, H), jnp.float32),
        scratch_shapes=[pltpu.VMEM((bg, rows, 4 * H), jnp.float32)],
        compiler_params=pltpu.CompilerParams(
            dimension_semantics=("parallel",)),
    )(ids, lens, table, wih, whh, b)


def kernel(maxlen, ids, lens, table, wih, whh, b):
    G = ids.shape[0]
    bg = 64
    while G % bg:
        bg //= 2
    return _run(ids, lens, table, wih, whh, b, bg=bg)


# speedup vs baseline: 1.4429x; 1.3260x over previous
from functools import partial

import jax
import jax.numpy as jnp
from jax import lax
from jax.experimental import pallas as pl
from jax.experimental.pallas import tpu as pltpu

H = 128
SUB = 8
LPAD = 32
NC = 128


def _lstm_kernel(ids_ref, lens_ref, table_ref, wih_ref, whh_ref, b_ref,
                 out_ref, *, bg, chains):
    B = bg * SUB
    BGC = bg // chains
    BC = BGC * SUB

    tw = (jnp.dot(table_ref[...], wih_ref[...],
                  preferred_element_type=jnp.float32)
          + b_ref[...]).astype(jnp.bfloat16)
    w_cat = jnp.concatenate([tw, whh_ref[...]], axis=0)

    iota = lax.broadcasted_iota(jnp.int32, (1, NC), 1)
    lens = lens_ref[...].reshape(B, 1)

    def step(t, carry):
        out = []
        for k in range(chains):
            h, c, hout = carry[k]
            idt = ids_ref[k * BGC:(k + 1) * BGC, pl.ds(t * SUB, SUB), :]
            onehot = jnp.where(idt.reshape(BC, 1) == iota,
                               1.0, 0.0).astype(jnp.bfloat16)
            lhs = jnp.concatenate([onehot, h.astype(jnp.bfloat16)], axis=1)
            gates = jnp.dot(lhs, w_cat, preferred_element_type=jnp.float32)
            sig = jax.nn.sigmoid(gates[:, :3 * H])
            g = jnp.tanh(gates[:, 3 * H:])
            i, f, o = sig[:, :H], sig[:, H:2 * H], sig[:, 2 * H:]
            c_new = f * c + i * g
            h_new = o * jnp.tanh(c_new)
            last = t == lens[k * BC:(k + 1) * BC] - 1
            out.append((h_new, c_new, jnp.where(last, h_new, hout)))
        return tuple(out)

    zeros = lambda: jnp.zeros((BC, H), jnp.float32)
    init = tuple((zeros(), zeros(), zeros()) for _ in range(chains))
    final = lax.fori_loop(0, LPAD, step, init, unroll=True)
    for k in range(chains):
        out_ref[k * BGC:(k + 1) * BGC, :, :] = final[k][2].reshape(BGC, SUB, H)


@partial(jax.jit, static_argnames=("bg",))
def _run(ids, lens, table, wih, whh, b, *, bg):
    G = ids.shape[0]
    GB = G // bg
    rows = LPAD * SUB
    return pl.pallas_call(
        partial(_lstm_kernel, bg=bg, chains=2),
        grid=(GB,),
        in_specs=[
            pl.BlockSpec((bg, rows, 1), lambda g: (g, 0, 0)),
            pl.BlockSpec((bg, SUB, 1), lambda g: (g, 0, 0)),
            pl.BlockSpec((NC, H), lambda g: (0, 0)),
            pl.BlockSpec((H, 4 * H), lambda g: (0, 0)),
            pl.BlockSpec((H, 4 * H), lambda g: (0, 0)),
            pl.BlockSpec((1, 4 * H), lambda g: (0, 0)),
        ],
        out_specs=pl.BlockSpec((bg, SUB, H), lambda g: (g, 0, 0)),
        out_shape=jax.ShapeDtypeStruct((G, SUB, H), jnp.float32),
        compiler_params=pltpu.CompilerParams(
            dimension_semantics=("parallel",)),
    )(ids, lens, table, wih, whh, b)


def kernel(maxlen, ids, lens, table, wih, whh, b):
    G = ids.shape[0]
    bg = 64
    while G % bg:
        bg //= 2
    return _run(ids, lens, table, wih, whh, b, bg=bg)


# speedup vs baseline: 1.4765x; 1.0232x over previous
from functools import partial

import jax
import jax.numpy as jnp
from jax import lax
from jax.experimental import pallas as pl
from jax.experimental.pallas import tpu as pltpu

H = 128
SUB = 8
LPAD = 32
NC = 128


def _lstm_kernel(ids_ref, lens_ref, table_ref, wih_ref, whh_ref, b_ref,
                 out_ref, *, bg, chains):
    B = bg * SUB
    BGC = bg // chains
    BC = BGC * SUB

    tw = (jnp.dot(table_ref[...], wih_ref[...],
                  preferred_element_type=jnp.float32)
          + b_ref[...]).astype(jnp.bfloat16)
    w_cat = jnp.concatenate([tw, whh_ref[...]], axis=0)

    iota = lax.broadcasted_iota(jnp.int32, (1, NC), 1)
    lens = lens_ref[...].reshape(B, 1)

    def step(t, carry):
        out = []
        for k in range(chains):
            h, c, hout = carry[k]
            idt = ids_ref[k * BGC:(k + 1) * BGC, pl.ds(t * SUB, SUB), :]
            onehot = jnp.where(idt.reshape(BC, 1) == iota,
                               1.0, 0.0).astype(jnp.bfloat16)
            lhs = jnp.concatenate([onehot, h.astype(jnp.bfloat16)], axis=1)
            gates = jnp.dot(lhs, w_cat, preferred_element_type=jnp.float32)
            sig = jax.nn.sigmoid(gates[:, :3 * H])
            g = jnp.tanh(gates[:, 3 * H:])
            i, f, o = sig[:, :H], sig[:, H:2 * H], sig[:, 2 * H:]
            c_new = f * c + i * g
            h_new = o * jnp.tanh(c_new)
            last = t == lens[k * BC:(k + 1) * BC] - 1
            out.append((h_new, c_new, jnp.where(last, h_new, hout)))
        return tuple(out)

    zeros = lambda: jnp.zeros((BC, H), jnp.float32)
    init = tuple((zeros(), zeros(), zeros()) for _ in range(chains))
    final = lax.fori_loop(0, LPAD, step, init, unroll=True)
    for k in range(chains):
        out_ref[k * BGC:(k + 1) * BGC, :, :] = final[k][2].reshape(BGC, SUB, H)


@partial(jax.jit, static_argnames=("bg",))
def _run(ids, lens, table, wih, whh, b, *, bg):
    G = ids.shape[0]
    GB = G // bg
    rows = LPAD * SUB
    return pl.pallas_call(
        partial(_lstm_kernel, bg=bg, chains=2),
        grid=(GB,),
        in_specs=[
            pl.BlockSpec((bg, rows, 1), lambda g: (g, 0, 0)),
            pl.BlockSpec((bg, SUB, 1), lambda g: (g, 0, 0)),
            pl.BlockSpec((NC, H), lambda g: (0, 0)),
            pl.BlockSpec((H, 4 * H), lambda g: (0, 0)),
            pl.BlockSpec((H, 4 * H), lambda g: (0, 0)),
            pl.BlockSpec((1, 4 * H), lambda g: (0, 0)),
        ],
        out_specs=pl.BlockSpec((bg, SUB, H), lambda g: (g, 0, 0)),
        out_shape=jax.ShapeDtypeStruct((G, SUB, H), jnp.float32),
        compiler_params=pltpu.CompilerParams(
            dimension_semantics=("parallel",)),
    )(ids, lens, table, wih, whh, b)


def kernel(maxlen, ids, lens, table, wih, whh, b):
    G = ids.shape[0]
    bg = 128
    while G % bg:
        bg //= 2
    return _run(ids, lens, table, wih, whh, b, bg=bg)
